# baseline (speedup 1.0000x reference)
import functools

import jax
import jax.numpy as jnp
from jax import lax
from jax.experimental import pallas as pl
from jax.experimental.pallas import tpu as pltpu
from jax.experimental.pallas import tpu_sc as plsc

N = 10000
E = 320000
D_IN = 128
D_HID = 128
D_OUT = 64

NC, NS = 2, 16
NP = 10240
SLICE = NP // NS
K = 128
PAD_NODE = NP - 1

TW = 2560
EP = TW * K
WF = TW // NS
WE = TW // (NC * NS)
CH = 80
NCH = WF // CH
NB = 5
DEG_FIRE = 20

_mesh = plsc.VectorSubcoreMesh(core_axis_name="c", subcore_axis_name="s")
_sc_params = pltpu.CompilerParams(use_tc_tiling_on_sc=False)


@functools.partial(
    pl.kernel,
    out_type=jax.ShapeDtypeStruct((NC, NP), jnp.float32),
    mesh=_mesh,
    scratch_types=[
        pltpu.VMEM((WE, K), jnp.int32),
        pltpu.VMEM((K,), jnp.float32),
        pltpu.VMEM((SLICE,), jnp.float32),
        pltpu.VMEM_SHARED((NP,), jnp.float32),
        pltpu.SemaphoreType.DMA,
    ],
    compiler_params=_sc_params,
)
def _deg_kernel(e_hbm, out_hbm, dst_v, ones_v, z_v, acc, sem):
    c = lax.axis_index("c")
    s = lax.axis_index("s")
    wid = c * NS + s

    @pl.loop(0, K, step=16)
    def _(i):
        ones_v[pl.ds(i, 16)] = jnp.ones((16,), jnp.float32)

    @pl.loop(0, SLICE, step=16)
    def _(i):
        z_v[pl.ds(i, 16)] = jnp.zeros((16,), jnp.float32)

    pltpu.sync_copy(e_hbm.at[1, pl.ds(wid * WE, WE)], dst_v)
    pltpu.sync_copy(z_v, acc.at[pl.ds(s * SLICE, SLICE)])
    plsc.subcore_barrier()

    @pl.loop(0, WE, step=DEG_FIRE)
    def _(w0):
        @pl.loop(0, DEG_FIRE)
        def _(i):
            pltpu.async_copy(ones_v, acc.at[dst_v.at[w0 + i]], sem, add=True)

        @pl.loop(0, DEG_FIRE)
        def _(i):
            pltpu.make_async_copy(ones_v, acc.at[dst_v.at[w0 + i]], sem).wait()

    plsc.subcore_barrier()
    pltpu.sync_copy(acc.at[pl.ds(s * SLICE, SLICE)],
                    out_hbm.at[c, pl.ds(s * SLICE, SLICE)])


def _make_prop(d2):
    @functools.partial(
        pl.kernel,
        out_type=jax.ShapeDtypeStruct((NC, NP, d2), jnp.float32),
        mesh=_mesh,
        scratch_types=[
            pltpu.VMEM((CH, K), jnp.int32),
            pltpu.VMEM((CH, K), jnp.int32),
            pltpu.VMEM((NB, K, d2), jnp.float32),
            pltpu.VMEM_SHARED((NP, d2), jnp.float32),
        ] + [pltpu.SemaphoreType.DMA] * NB,
        compiler_params=_sc_params,
    )
    def _prop(u_hbm, e_hbm, out_hbm, src_v, dst_v, rows_v, acc, *sems):
        c = lax.axis_index("c")
        s = lax.axis_index("s")
        pltpu.sync_copy(u_hbm.at[c, pl.ds(s * SLICE, SLICE)],
                        acc.at[pl.ds(s * SLICE, SLICE)])
        plsc.subcore_barrier()

        ub = u_hbm.at[c]

        def gather(w, b):
            pltpu.async_copy(ub.at[src_v.at[w]], rows_v.at[b], sems[b])

        def wait_scatter(w, b):
            pltpu.make_async_copy(ub.at[src_v.at[w]], rows_v.at[b],
                                  sems[b]).wait()
            pltpu.sync_copy(rows_v.at[b], acc.at[dst_v.at[w]], add=True)

        @pl.loop(0, NCH)
        def _(ch):
            pltpu.sync_copy(e_hbm.at[0, pl.ds(s * WF + ch * CH, CH)], src_v)
            pltpu.sync_copy(e_hbm.at[1, pl.ds(s * WF + ch * CH, CH)], dst_v)

            for b in range(NB):
                gather(b, b)

            @pl.loop(0, CH // NB - 1)
            def _(j):
                for b in range(NB):
                    w = j * NB + b
                    wait_scatter(w, b)
                    gather(w + NB, b)

            for b in range(NB):
                wait_scatter(CH - NB + b, b)

        plsc.subcore_barrier()
        pltpu.sync_copy(acc.at[pl.ds(s * SLICE, SLICE)],
                        out_hbm.at[c, pl.ds(s * SLICE, SLICE)])

    return _prop


_prop128 = _make_prop(D_HID // NC)


@functools.partial(
    pl.kernel,
    out_type=jax.ShapeDtypeStruct((NC, NP, D_OUT), jnp.float32),
    mesh=_mesh,
    scratch_types=[
        pltpu.VMEM((WE, K), jnp.int32),
        pltpu.VMEM((WE, K), jnp.int32),
        pltpu.VMEM((NB, K, D_OUT), jnp.float32),
        pltpu.VMEM_SHARED((NP, D_OUT), jnp.float32),
    ] + [pltpu.SemaphoreType.DMA] * NB,
    compiler_params=_sc_params,
)
def _prop64(u_hbm, e_hbm, out_hbm, src_v, dst_v, rows_v, acc, *sems):
    c = lax.axis_index("c")
    s = lax.axis_index("s")
    wid = c * NS + s
    pltpu.sync_copy(e_hbm.at[0, pl.ds(wid * WE, WE)], src_v)
    pltpu.sync_copy(e_hbm.at[1, pl.ds(wid * WE, WE)], dst_v)
    pltpu.sync_copy(u_hbm.at[pl.ds(s * SLICE, SLICE)],
                    acc.at[pl.ds(s * SLICE, SLICE)])
    plsc.subcore_barrier()

    def gather(w, b):
        pltpu.async_copy(u_hbm.at[src_v.at[w]], rows_v.at[b], sems[b])

    def wait_scatter(w, b):
        pltpu.make_async_copy(u_hbm.at[src_v.at[w]], rows_v.at[b],
                              sems[b]).wait()
        pltpu.sync_copy(rows_v.at[b], acc.at[dst_v.at[w]], add=True)

    for b in range(NB):
        gather(b, b)

    @pl.loop(0, WE // NB - 1)
    def _(j):
        for b in range(NB):
            w = j * NB + b
            wait_scatter(w, b)
            gather(w + NB, b)

    for b in range(NB):
        wait_scatter(WE - NB + b, b)

    plsc.subcore_barrier()
    pltpu.sync_copy(acc.at[pl.ds(s * SLICE, SLICE)],
                    out_hbm.at[c, pl.ds(s * SLICE, SLICE)])


_BLK = 2048
_GRID = NP // _BLK
_tc_params = pltpu.CompilerParams(dimension_semantics=("parallel",))
_H2 = D_HID // NC
_O2 = D_OUT // NC


def _dinv(degp_ref):
    i = pl.program_id(0)
    deg = degp_ref[0, pl.ds(i * _BLK, _BLK)] + degp_ref[1, pl.ds(i * _BLK, _BLK)] + 1.0
    return lax.rsqrt(deg)[:, None]


def _layer1_body(degp_ref, x_ref, w1_ref, u_ref):
    t1 = lax.dot_general(x_ref[...], w1_ref[...], (((1,), (1,)), ((), ())),
                         preferred_element_type=jnp.float32)
    scaled = t1 * _dinv(degp_ref)
    u_ref[0] = scaled[:, :_H2]
    u_ref[1] = scaled[:, _H2:]


def _layer2_body(degp_ref, parts_ref, b1_ref, w2_ref, u2_ref):
    dinv = _dinv(degp_ref)
    agg = jnp.concatenate([parts_ref[0], parts_ref[1]], axis=-1)
    h = jnp.maximum(agg * dinv + b1_ref[...], 0.0)
    t2 = lax.dot_general(h, w2_ref[...], (((1,), (1,)), ((), ())),
                         preferred_element_type=jnp.float32)
    u2_ref[...] = t2 * dinv


def _final_body(degp_ref, parts_ref, u2_ref, b2_ref, o_ref):
    agg = parts_ref[0] + parts_ref[1] - u2_ref[...]
    o_ref[...] = agg * _dinv(degp_ref) + b2_ref[...]


def _full(shape):
    return pl.BlockSpec(shape, lambda i: tuple(0 for _ in shape))


def _rows(d):
    return pl.BlockSpec((_BLK, d), lambda i: (i, 0))


_degs = pl.BlockSpec((NC, NP), lambda i: (0, 0))


def _parts(d2):
    return pl.BlockSpec((NC, _BLK, d2), lambda i: (0, i, 0))


def kernel(x, edge_index, W1, b1, W2, b2):
    e4 = jnp.pad(edge_index, ((0, 0), (0, EP - E)),
                 constant_values=PAD_NODE).reshape(2, TW, K)
    xp = jnp.pad(x, ((0, NP - N), (0, 0)))
    b1r = b1.reshape(1, D_HID)
    b2r = b2.reshape(1, D_OUT)

    degp = _deg_kernel(e4)

    u1 = pl.pallas_call(
        _layer1_body,
        grid=(_GRID,),
        in_specs=[_degs, _rows(D_IN), _full((D_HID, D_IN))],
        out_specs=_parts(_H2),
        out_shape=jax.ShapeDtypeStruct((NC, NP, _H2), jnp.float32),
        compiler_params=_tc_params,
    )(degp, xp, W1)

    parts1 = _prop128(u1, e4)

    u2 = pl.pallas_call(
        _layer2_body,
        grid=(_GRID,),
        in_specs=[_degs, _parts(_H2), _full((1, D_HID)),
                  _full((D_OUT, D_HID))],
        out_specs=_rows(D_OUT),
        out_shape=jax.ShapeDtypeStruct((NP, D_OUT), jnp.float32),
        compiler_params=_tc_params,
    )(degp, parts1, b1r, W2)

    parts2 = _prop64(u2, e4)

    outp = pl.pallas_call(
        _final_body,
        grid=(_GRID,),
        in_specs=[_degs,
                  pl.BlockSpec((NC, _BLK, D_OUT), lambda i: (0, i, 0)),
                  _rows(D_OUT), _full((1, D_OUT))],
        out_specs=_rows(D_OUT),
        out_shape=jax.ShapeDtypeStruct((NP, D_OUT), jnp.float32),
        compiler_params=_tc_params,
    )(degp, parts2, u2, b2r)

    return outp[:N]


# speedup vs baseline: 2.5976x; 2.5976x over previous
import functools

import jax
import jax.numpy as jnp
from jax import lax
from jax.experimental import pallas as pl
from jax.experimental.pallas import tpu as pltpu
from jax.experimental.pallas import tpu_sc as plsc

N = 10000
E = 320000
D_IN = 128
D_HID = 128
D_OUT = 64

NC, NS = 2, 16
NP = 10240
SLICE = NP // NS
K = 128
PAD_NODE = NP - 1

TW = 2560
EP = TW * K
WF = TW // NS
WE = TW // (NC * NS)
CH = 80
NCH = WF // CH
NB = 5
DEG_FIRE = 20

_mesh = plsc.VectorSubcoreMesh(core_axis_name="c", subcore_axis_name="s")
_sc_params = pltpu.CompilerParams(use_tc_tiling_on_sc=False)


@functools.partial(
    pl.kernel,
    out_type=jax.ShapeDtypeStruct((NC, NP), jnp.float32),
    mesh=_mesh,
    scratch_types=[
        pltpu.VMEM((WE, K), jnp.int32),
        pltpu.VMEM((K,), jnp.float32),
        pltpu.VMEM((SLICE,), jnp.float32),
        pltpu.VMEM_SHARED((NP,), jnp.float32),
        pltpu.SemaphoreType.DMA,
    ],
    compiler_params=_sc_params,
)
def _deg_kernel(e_hbm, out_hbm, dst_v, ones_v, z_v, acc, sem):
    c = lax.axis_index("c")
    s = lax.axis_index("s")
    wid = c * NS + s

    @pl.loop(0, K, step=16)
    def _(i):
        ones_v[pl.ds(i, 16)] = jnp.ones((16,), jnp.float32)

    @pl.loop(0, SLICE, step=16)
    def _(i):
        z_v[pl.ds(i, 16)] = jnp.zeros((16,), jnp.float32)

    pltpu.sync_copy(e_hbm.at[1, pl.ds(wid * WE, WE)], dst_v)
    pltpu.sync_copy(z_v, acc.at[pl.ds(s * SLICE, SLICE)])
    plsc.subcore_barrier()

    @pl.loop(0, WE, step=DEG_FIRE)
    def _(w0):
        @pl.loop(0, DEG_FIRE)
        def _(i):
            pltpu.async_copy(ones_v, acc.at[dst_v.at[w0 + i]], sem, add=True)

        @pl.loop(0, DEG_FIRE)
        def _(i):
            pltpu.make_async_copy(ones_v, acc.at[dst_v.at[w0 + i]], sem).wait()

    plsc.subcore_barrier()
    pltpu.sync_copy(acc.at[pl.ds(s * SLICE, SLICE)],
                    out_hbm.at[c, pl.ds(s * SLICE, SLICE)])


def _make_prop(d2):
    @functools.partial(
        pl.kernel,
        out_type=jax.ShapeDtypeStruct((NC, NP, d2), jnp.float32),
        mesh=_mesh,
        scratch_types=[
            pltpu.VMEM((CH, K), jnp.int32),
            pltpu.VMEM((CH, K), jnp.int32),
            pltpu.VMEM((NB, K, d2), jnp.float32),
            pltpu.VMEM_SHARED((NP, d2), jnp.float32),
        ] + [pltpu.SemaphoreType.DMA] * NB,
        compiler_params=_sc_params,
    )
    def _prop(u_hbm, e_hbm, out_hbm, src_v, dst_v, rows_v, acc, *sems):
        c = lax.axis_index("c")
        s = lax.axis_index("s")
        pltpu.sync_copy(u_hbm.at[c, pl.ds(s * SLICE, SLICE)],
                        acc.at[pl.ds(s * SLICE, SLICE)])
        plsc.subcore_barrier()

        ub = u_hbm.at[c]

        def gather(w, b):
            pltpu.async_copy(ub.at[src_v.at[w]], rows_v.at[b], sems[b])

        def wait_scatter(w, b):
            pltpu.make_async_copy(ub.at[src_v.at[w]], rows_v.at[b],
                                  sems[b]).wait()
            pltpu.sync_copy(rows_v.at[b], acc.at[dst_v.at[w]], add=True)

        @pl.loop(0, NCH)
        def _(ch):
            pltpu.sync_copy(e_hbm.at[0, pl.ds(s * WF + ch * CH, CH)], src_v)
            pltpu.sync_copy(e_hbm.at[1, pl.ds(s * WF + ch * CH, CH)], dst_v)

            for b in range(NB):
                gather(b, b)

            @pl.loop(0, CH // NB - 1)
            def _(j):
                for b in range(NB):
                    w = j * NB + b
                    wait_scatter(w, b)
                    gather(w + NB, b)

            for b in range(NB):
                wait_scatter(CH - NB + b, b)

        plsc.subcore_barrier()
        pltpu.sync_copy(acc.at[pl.ds(s * SLICE, SLICE)],
                        out_hbm.at[c, pl.ds(s * SLICE, SLICE)])

    return _prop


_prop128 = _make_prop(D_HID // NC)


@functools.partial(
    pl.kernel,
    out_type=jax.ShapeDtypeStruct((NC, NP, D_OUT), jnp.float32),
    mesh=_mesh,
    scratch_types=[
        pltpu.VMEM((WE, K), jnp.int32),
        pltpu.VMEM((WE, K), jnp.int32),
        pltpu.VMEM((NB, K, D_OUT), jnp.float32),
        pltpu.VMEM_SHARED((NP, D_OUT), jnp.float32),
    ] + [pltpu.SemaphoreType.DMA] * NB,
    compiler_params=_sc_params,
)
def _prop64(u_hbm, e_hbm, out_hbm, src_v, dst_v, rows_v, acc, *sems):
    c = lax.axis_index("c")
    s = lax.axis_index("s")
    wid = c * NS + s
    pltpu.sync_copy(e_hbm.at[0, pl.ds(wid * WE, WE)], src_v)
    pltpu.sync_copy(e_hbm.at[1, pl.ds(wid * WE, WE)], dst_v)
    pltpu.sync_copy(u_hbm.at[pl.ds(s * SLICE, SLICE)],
                    acc.at[pl.ds(s * SLICE, SLICE)])
    plsc.subcore_barrier()

    def gather(w, b):
        pltpu.async_copy(u_hbm.at[src_v.at[w]], rows_v.at[b], sems[b])

    def wait_scatter(w, b):
        pltpu.make_async_copy(u_hbm.at[src_v.at[w]], rows_v.at[b],
                              sems[b]).wait()
        pltpu.sync_copy(rows_v.at[b], acc.at[dst_v.at[w]], add=True)

    for b in range(NB):
        gather(b, b)

    @pl.loop(0, WE // NB - 1)
    def _(j):
        for b in range(NB):
            w = j * NB + b
            wait_scatter(w, b)
            gather(w + NB, b)

    for b in range(NB):
        wait_scatter(WE - NB + b, b)

    plsc.subcore_barrier()
    pltpu.sync_copy(acc.at[pl.ds(s * SLICE, SLICE)],
                    out_hbm.at[c, pl.ds(s * SLICE, SLICE)])


_BLK = 2048
_GRID = NP // _BLK
_tc_params = pltpu.CompilerParams(dimension_semantics=("parallel",))
_H2 = D_HID // NC
_O2 = D_OUT // NC


def _dinv(degp_ref):
    i = pl.program_id(0)
    deg = degp_ref[0, pl.ds(i * _BLK, _BLK)] + degp_ref[1, pl.ds(i * _BLK, _BLK)] + 1.0
    return lax.rsqrt(deg)[:, None]


def _layer1_body(degp_ref, x_ref, w1_ref, u_ref):
    t1 = lax.dot_general(x_ref[...], w1_ref[...], (((1,), (1,)), ((), ())),
                         preferred_element_type=jnp.float32)
    scaled = t1 * _dinv(degp_ref)
    u_ref[0] = scaled[:, :_H2]
    u_ref[1] = scaled[:, _H2:]


def _layer2_body(degp_ref, parts_ref, b1_ref, w2_ref, u2_ref):
    dinv = _dinv(degp_ref)
    agg = jnp.concatenate([parts_ref[0], parts_ref[1]], axis=-1)
    h = jnp.maximum(agg * dinv + b1_ref[...], 0.0)
    t2 = lax.dot_general(h, w2_ref[...], (((1,), (1,)), ((), ())),
                         preferred_element_type=jnp.float32)
    u2_ref[...] = t2 * dinv


def _final_body(degp_ref, parts_ref, u2_ref, b2_ref, o_ref):
    agg = parts_ref[0] + parts_ref[1] - u2_ref[...]
    o_ref[...] = agg * _dinv(degp_ref) + b2_ref[...]


def _full(shape):
    return pl.BlockSpec(shape, lambda i: tuple(0 for _ in shape))


def _rows(d):
    return pl.BlockSpec((_BLK, d), lambda i: (i, 0))


_degs = pl.BlockSpec((NC, NP), lambda i: (0, 0))


def _parts(d2):
    return pl.BlockSpec((NC, _BLK, d2), lambda i: (0, i, 0))


def kernel(x, edge_index, W1, b1, W2, b2):
    pad_idx = N + jnp.arange(EP - E, dtype=edge_index.dtype) % (NP - N)
    e4 = jnp.concatenate(
        [edge_index, jnp.broadcast_to(pad_idx, (2, EP - E))], axis=1
    ).reshape(2, TW, K)
    xp = jnp.pad(x, ((0, NP - N), (0, 0)))
    b1r = b1.reshape(1, D_HID)
    b2r = b2.reshape(1, D_OUT)

    degp = _deg_kernel(e4)

    u1 = pl.pallas_call(
        _layer1_body,
        grid=(_GRID,),
        in_specs=[_degs, _rows(D_IN), _full((D_HID, D_IN))],
        out_specs=_parts(_H2),
        out_shape=jax.ShapeDtypeStruct((NC, NP, _H2), jnp.float32),
        compiler_params=_tc_params,
    )(degp, xp, W1)

    parts1 = _prop128(u1, e4)

    u2 = pl.pallas_call(
        _layer2_body,
        grid=(_GRID,),
        in_specs=[_degs, _parts(_H2), _full((1, D_HID)),
                  _full((D_OUT, D_HID))],
        out_specs=_rows(D_OUT),
        out_shape=jax.ShapeDtypeStruct((NP, D_OUT), jnp.float32),
        compiler_params=_tc_params,
    )(degp, parts1, b1r, W2)

    parts2 = _prop64(u2, e4)

    outp = pl.pallas_call(
        _final_body,
        grid=(_GRID,),
        in_specs=[_degs,
                  pl.BlockSpec((NC, _BLK, D_OUT), lambda i: (0, i, 0)),
                  _rows(D_OUT), _full((1, D_OUT))],
        out_specs=_rows(D_OUT),
        out_shape=jax.ShapeDtypeStruct((NP, D_OUT), jnp.float32),
        compiler_params=_tc_params,
    )(degp, parts2, u2, b2r)

    return outp[:N]


# speedup vs baseline: 2.6858x; 1.0339x over previous
import functools

import jax
import jax.numpy as jnp
from jax import lax
from jax.experimental import pallas as pl
from jax.experimental.pallas import tpu as pltpu
from jax.experimental.pallas import tpu_sc as plsc

N = 10000
E = 320000
D_IN = 128
D_HID = 128
D_OUT = 64

NC, NS = 2, 16
NP = 10240
SLICE = NP // NS
K = 80

TW = E // K
WF = TW // NS
WE = TW // (NC * NS)
CH = 125
NCH = WF // CH
NB = 5
DEG_FIRE = 25

_mesh = plsc.VectorSubcoreMesh(core_axis_name="c", subcore_axis_name="s")
_sc_params = pltpu.CompilerParams(use_tc_tiling_on_sc=False)


@functools.partial(
    pl.kernel,
    out_type=jax.ShapeDtypeStruct((NC, NP), jnp.float32),
    mesh=_mesh,
    scratch_types=[
        pltpu.VMEM((WE, K), jnp.int32),
        pltpu.VMEM((K,), jnp.float32),
        pltpu.VMEM((SLICE,), jnp.float32),
        pltpu.VMEM_SHARED((NP,), jnp.float32),
        pltpu.SemaphoreType.DMA,
    ],
    compiler_params=_sc_params,
)
def _deg_kernel(e_hbm, out_hbm, dst_v, ones_v, z_v, acc, sem):
    c = lax.axis_index("c")
    s = lax.axis_index("s")
    wid = c * NS + s

    @pl.loop(0, K, step=16)
    def _(i):
        ones_v[pl.ds(i, 16)] = jnp.ones((16,), jnp.float32)

    @pl.loop(0, SLICE, step=16)
    def _(i):
        z_v[pl.ds(i, 16)] = jnp.zeros((16,), jnp.float32)

    pltpu.sync_copy(e_hbm.at[1, pl.ds(wid * WE, WE)], dst_v)
    pltpu.sync_copy(z_v, acc.at[pl.ds(s * SLICE, SLICE)])
    plsc.subcore_barrier()

    @pl.loop(0, WE, step=DEG_FIRE)
    def _(w0):
        @pl.loop(0, DEG_FIRE)
        def _(i):
            pltpu.async_copy(ones_v, acc.at[dst_v.at[w0 + i]], sem, add=True)

        @pl.loop(0, DEG_FIRE)
        def _(i):
            pltpu.make_async_copy(ones_v, acc.at[dst_v.at[w0 + i]], sem).wait()

    plsc.subcore_barrier()
    pltpu.sync_copy(acc.at[pl.ds(s * SLICE, SLICE)],
                    out_hbm.at[c, pl.ds(s * SLICE, SLICE)])


def _make_prop(d2):
    @functools.partial(
        pl.kernel,
        out_type=jax.ShapeDtypeStruct((NC, NP, d2), jnp.float32),
        mesh=_mesh,
        scratch_types=[
            pltpu.VMEM((CH, K), jnp.int32),
            pltpu.VMEM((CH, K), jnp.int32),
            pltpu.VMEM((NB, K, d2), jnp.float32),
            pltpu.VMEM_SHARED((NP, d2), jnp.float32),
        ] + [pltpu.SemaphoreType.DMA] * NB,
        compiler_params=_sc_params,
    )
    def _prop(u_hbm, e_hbm, out_hbm, src_v, dst_v, rows_v, acc, *sems):
        c = lax.axis_index("c")
        s = lax.axis_index("s")
        pltpu.sync_copy(u_hbm.at[c, pl.ds(s * SLICE, SLICE)],
                        acc.at[pl.ds(s * SLICE, SLICE)])
        plsc.subcore_barrier()

        ub = u_hbm.at[c]

        def gather(w, b):
            pltpu.async_copy(ub.at[src_v.at[w]], rows_v.at[b], sems[b])

        def wait_scatter(w, b):
            pltpu.make_async_copy(ub.at[src_v.at[w]], rows_v.at[b],
                                  sems[b]).wait()
            pltpu.sync_copy(rows_v.at[b], acc.at[dst_v.at[w]], add=True)

        @pl.loop(0, NCH)
        def _(ch):
            pltpu.sync_copy(e_hbm.at[0, pl.ds(s * WF + ch * CH, CH)], src_v)
            pltpu.sync_copy(e_hbm.at[1, pl.ds(s * WF + ch * CH, CH)], dst_v)

            for b in range(NB):
                gather(b, b)

            @pl.loop(0, CH // NB - 1)
            def _(j):
                for b in range(NB):
                    w = j * NB + b
                    wait_scatter(w, b)
                    gather(w + NB, b)

            for b in range(NB):
                wait_scatter(CH - NB + b, b)

        plsc.subcore_barrier()
        pltpu.sync_copy(acc.at[pl.ds(s * SLICE, SLICE)],
                        out_hbm.at[c, pl.ds(s * SLICE, SLICE)])

    return _prop


_prop128 = _make_prop(D_HID // NC)


@functools.partial(
    pl.kernel,
    out_type=jax.ShapeDtypeStruct((NC, NP, D_OUT), jnp.float32),
    mesh=_mesh,
    scratch_types=[
        pltpu.VMEM((WE, K), jnp.int32),
        pltpu.VMEM((WE, K), jnp.int32),
        pltpu.VMEM((NB, K, D_OUT), jnp.float32),
        pltpu.VMEM_SHARED((NP, D_OUT), jnp.float32),
    ] + [pltpu.SemaphoreType.DMA] * NB,
    compiler_params=_sc_params,
)
def _prop64(u_hbm, e_hbm, out_hbm, src_v, dst_v, rows_v, acc, *sems):
    c = lax.axis_index("c")
    s = lax.axis_index("s")
    wid = c * NS + s
    pltpu.sync_copy(e_hbm.at[0, pl.ds(wid * WE, WE)], src_v)
    pltpu.sync_copy(e_hbm.at[1, pl.ds(wid * WE, WE)], dst_v)
    pltpu.sync_copy(u_hbm.at[pl.ds(s * SLICE, SLICE)],
                    acc.at[pl.ds(s * SLICE, SLICE)])
    plsc.subcore_barrier()

    def gather(w, b):
        pltpu.async_copy(u_hbm.at[src_v.at[w]], rows_v.at[b], sems[b])

    def wait_scatter(w, b):
        pltpu.make_async_copy(u_hbm.at[src_v.at[w]], rows_v.at[b],
                              sems[b]).wait()
        pltpu.sync_copy(rows_v.at[b], acc.at[dst_v.at[w]], add=True)

    for b in range(NB):
        gather(b, b)

    @pl.loop(0, WE // NB - 1)
    def _(j):
        for b in range(NB):
            w = j * NB + b
            wait_scatter(w, b)
            gather(w + NB, b)

    for b in range(NB):
        wait_scatter(WE - NB + b, b)

    plsc.subcore_barrier()
    pltpu.sync_copy(acc.at[pl.ds(s * SLICE, SLICE)],
                    out_hbm.at[c, pl.ds(s * SLICE, SLICE)])


_BLK = 2048
_GRID = NP // _BLK
_tc_params = pltpu.CompilerParams(dimension_semantics=("parallel",))
_H2 = D_HID // NC
_O2 = D_OUT // NC


def _dinv(degp_ref):
    i = pl.program_id(0)
    deg = degp_ref[0, pl.ds(i * _BLK, _BLK)] + degp_ref[1, pl.ds(i * _BLK, _BLK)] + 1.0
    return lax.rsqrt(deg)[:, None]


def _layer1_body(degp_ref, x_ref, w1_ref, u_ref):
    t1 = lax.dot_general(x_ref[...], w1_ref[...], (((1,), (1,)), ((), ())),
                         preferred_element_type=jnp.float32)
    scaled = t1 * _dinv(degp_ref)
    u_ref[0] = scaled[:, :_H2]
    u_ref[1] = scaled[:, _H2:]


def _layer2_body(degp_ref, parts_ref, b1_ref, w2_ref, u2_ref):
    dinv = _dinv(degp_ref)
    agg = jnp.concatenate([parts_ref[0], parts_ref[1]], axis=-1)
    h = jnp.maximum(agg * dinv + b1_ref[...], 0.0)
    t2 = lax.dot_general(h, w2_ref[...], (((1,), (1,)), ((), ())),
                         preferred_element_type=jnp.float32)
    u2_ref[...] = t2 * dinv


def _final_body(degp_ref, parts_ref, u2_ref, b2_ref, o_ref):
    agg = parts_ref[0] + parts_ref[1] - u2_ref[...]
    o_ref[...] = agg * _dinv(degp_ref) + b2_ref[...]


def _full(shape):
    return pl.BlockSpec(shape, lambda i: tuple(0 for _ in shape))


def _rows(d):
    return pl.BlockSpec((_BLK, d), lambda i: (i, 0))


_degs = pl.BlockSpec((NC, NP), lambda i: (0, 0))


def _parts(d2):
    return pl.BlockSpec((NC, _BLK, d2), lambda i: (0, i, 0))


def kernel(x, edge_index, W1, b1, W2, b2):
    e4 = edge_index.reshape(2, TW, K)
    xp = jnp.pad(x, ((0, NP - N), (0, 0)))
    b1r = b1.reshape(1, D_HID)
    b2r = b2.reshape(1, D_OUT)

    degp = _deg_kernel(e4)

    u1 = pl.pallas_call(
        _layer1_body,
        grid=(_GRID,),
        in_specs=[_degs, _rows(D_IN), _full((D_HID, D_IN))],
        out_specs=_parts(_H2),
        out_shape=jax.ShapeDtypeStruct((NC, NP, _H2), jnp.float32),
        compiler_params=_tc_params,
    )(degp, xp, W1)

    parts1 = _prop128(u1, e4)

    u2 = pl.pallas_call(
        _layer2_body,
        grid=(_GRID,),
        in_specs=[_degs, _parts(_H2), _full((1, D_HID)),
                  _full((D_OUT, D_HID))],
        out_specs=_rows(D_OUT),
        out_shape=jax.ShapeDtypeStruct((NP, D_OUT), jnp.float32),
        compiler_params=_tc_params,
    )(degp, parts1, b1r, W2)

    parts2 = _prop64(u2, e4)

    outp = pl.pallas_call(
        _final_body,
        grid=(_GRID,),
        in_specs=[_degs,
                  pl.BlockSpec((NC, _BLK, D_OUT), lambda i: (0, i, 0)),
                  _rows(D_OUT), _full((1, D_OUT))],
        out_specs=_rows(D_OUT),
        out_shape=jax.ShapeDtypeStruct((NP, D_OUT), jnp.float32),
        compiler_params=_tc_params,
    )(degp, parts2, u2, b2r)

    return outp[:N]


# speedup vs baseline: 2.7141x; 1.0106x over previous
import functools

import jax
import jax.numpy as jnp
from jax import lax
from jax.experimental import pallas as pl
from jax.experimental.pallas import tpu as pltpu
from jax.experimental.pallas import tpu_sc as plsc

N = 10000
E = 320000
D_IN = 128
D_HID = 128
D_OUT = 64

NC, NS = 2, 16
NP = 10240
SLICE = NP // NS
K = 80

TW = E // K
WF = TW // NS
WE = TW // (NC * NS)
CH = 250
NCH = WF // CH
NB = 5
DEG_FIRE = 25

_mesh = plsc.VectorSubcoreMesh(core_axis_name="c", subcore_axis_name="s")
_sc_params = pltpu.CompilerParams(use_tc_tiling_on_sc=False)


@functools.partial(
    pl.kernel,
    out_type=jax.ShapeDtypeStruct((NC, NP), jnp.float32),
    mesh=_mesh,
    scratch_types=[
        pltpu.VMEM((WE, K), jnp.int32),
        pltpu.VMEM((K,), jnp.float32),
        pltpu.VMEM((SLICE,), jnp.float32),
        pltpu.VMEM_SHARED((NP,), jnp.float32),
        pltpu.SemaphoreType.DMA,
    ],
    compiler_params=_sc_params,
)
def _deg_kernel(e_hbm, out_hbm, dst_v, ones_v, z_v, acc, sem):
    c = lax.axis_index("c")
    s = lax.axis_index("s")
    wid = c * NS + s

    @pl.loop(0, K, step=16)
    def _(i):
        ones_v[pl.ds(i, 16)] = jnp.ones((16,), jnp.float32)

    @pl.loop(0, SLICE, step=16)
    def _(i):
        z_v[pl.ds(i, 16)] = jnp.zeros((16,), jnp.float32)

    pltpu.sync_copy(e_hbm.at[1, pl.ds(wid * WE, WE)], dst_v)
    pltpu.sync_copy(z_v, acc.at[pl.ds(s * SLICE, SLICE)])
    plsc.subcore_barrier()

    @pl.loop(0, WE, step=DEG_FIRE)
    def _(w0):
        @pl.loop(0, DEG_FIRE)
        def _(i):
            pltpu.async_copy(ones_v, acc.at[dst_v.at[w0 + i]], sem, add=True)

        @pl.loop(0, DEG_FIRE)
        def _(i):
            pltpu.make_async_copy(ones_v, acc.at[dst_v.at[w0 + i]], sem).wait()

    plsc.subcore_barrier()
    pltpu.sync_copy(acc.at[pl.ds(s * SLICE, SLICE)],
                    out_hbm.at[c, pl.ds(s * SLICE, SLICE)])


def _make_prop(d2):
    @functools.partial(
        pl.kernel,
        out_type=jax.ShapeDtypeStruct((NC, NP, d2), jnp.float32),
        mesh=_mesh,
        scratch_types=[
            pltpu.VMEM((CH, K), jnp.int32),
            pltpu.VMEM((CH, K), jnp.int32),
            pltpu.VMEM((NB, K, d2), jnp.float32),
            pltpu.VMEM_SHARED((NP, d2), jnp.float32),
        ] + [pltpu.SemaphoreType.DMA] * NB,
        compiler_params=_sc_params,
    )
    def _prop(u_hbm, e_hbm, out_hbm, src_v, dst_v, rows_v, acc, *sems):
        c = lax.axis_index("c")
        s = lax.axis_index("s")
        pltpu.sync_copy(u_hbm.at[c, pl.ds(s * SLICE, SLICE)],
                        acc.at[pl.ds(s * SLICE, SLICE)])
        plsc.subcore_barrier()

        ub = u_hbm.at[c]

        def gather(w, b):
            pltpu.async_copy(ub.at[src_v.at[w]], rows_v.at[b], sems[b])

        def wait_scatter(w, b):
            pltpu.make_async_copy(ub.at[src_v.at[w]], rows_v.at[b],
                                  sems[b]).wait()
            pltpu.sync_copy(rows_v.at[b], acc.at[dst_v.at[w]], add=True)

        @pl.loop(0, NCH)
        def _(ch):
            pltpu.sync_copy(e_hbm.at[0, pl.ds(s * WF + ch * CH, CH)], src_v)
            pltpu.sync_copy(e_hbm.at[1, pl.ds(s * WF + ch * CH, CH)], dst_v)

            for b in range(NB):
                gather(b, b)

            @pl.loop(0, CH // NB - 1)
            def _(j):
                for b in range(NB):
                    w = j * NB + b
                    wait_scatter(w, b)
                    gather(w + NB, b)

            for b in range(NB):
                wait_scatter(CH - NB + b, b)

        plsc.subcore_barrier()
        pltpu.sync_copy(acc.at[pl.ds(s * SLICE, SLICE)],
                        out_hbm.at[c, pl.ds(s * SLICE, SLICE)])

    return _prop


_prop128 = _make_prop(D_HID // NC)


@functools.partial(
    pl.kernel,
    out_type=jax.ShapeDtypeStruct((NC, NP, D_OUT), jnp.float32),
    mesh=_mesh,
    scratch_types=[
        pltpu.VMEM((WE, K), jnp.int32),
        pltpu.VMEM((WE, K), jnp.int32),
        pltpu.VMEM((NB, K, D_OUT), jnp.float32),
        pltpu.VMEM_SHARED((NP, D_OUT), jnp.float32),
    ] + [pltpu.SemaphoreType.DMA] * NB,
    compiler_params=_sc_params,
)
def _prop64(u_hbm, e_hbm, out_hbm, src_v, dst_v, rows_v, acc, *sems):
    c = lax.axis_index("c")
    s = lax.axis_index("s")
    wid = c * NS + s
    pltpu.sync_copy(e_hbm.at[0, pl.ds(wid * WE, WE)], src_v)
    pltpu.sync_copy(e_hbm.at[1, pl.ds(wid * WE, WE)], dst_v)
    pltpu.sync_copy(u_hbm.at[pl.ds(s * SLICE, SLICE)],
                    acc.at[pl.ds(s * SLICE, SLICE)])
    plsc.subcore_barrier()

    def gather(w, b):
        pltpu.async_copy(u_hbm.at[src_v.at[w]], rows_v.at[b], sems[b])

    def wait_scatter(w, b):
        pltpu.make_async_copy(u_hbm.at[src_v.at[w]], rows_v.at[b],
                              sems[b]).wait()
        pltpu.sync_copy(rows_v.at[b], acc.at[dst_v.at[w]], add=True)

    for b in range(NB):
        gather(b, b)

    @pl.loop(0, WE // NB - 1)
    def _(j):
        for b in range(NB):
            w = j * NB + b
            wait_scatter(w, b)
            gather(w + NB, b)

    for b in range(NB):
        wait_scatter(WE - NB + b, b)

    plsc.subcore_barrier()
    pltpu.sync_copy(acc.at[pl.ds(s * SLICE, SLICE)],
                    out_hbm.at[c, pl.ds(s * SLICE, SLICE)])


_BLK = 2048
_GRID = NP // _BLK
_tc_params = pltpu.CompilerParams(dimension_semantics=("parallel",))
_H2 = D_HID // NC
_O2 = D_OUT // NC


def _dinv(degp_ref):
    i = pl.program_id(0)
    deg = degp_ref[0, pl.ds(i * _BLK, _BLK)] + degp_ref[1, pl.ds(i * _BLK, _BLK)] + 1.0
    return lax.rsqrt(deg)[:, None]


def _layer1_body(degp_ref, x_ref, w1_ref, u_ref):
    t1 = lax.dot_general(x_ref[...], w1_ref[...], (((1,), (1,)), ((), ())),
                         preferred_element_type=jnp.float32)
    scaled = t1 * _dinv(degp_ref)
    u_ref[0] = scaled[:, :_H2]
    u_ref[1] = scaled[:, _H2:]


def _layer2_body(degp_ref, parts_ref, b1_ref, w2_ref, u2_ref):
    dinv = _dinv(degp_ref)
    agg = jnp.concatenate([parts_ref[0], parts_ref[1]], axis=-1)
    h = jnp.maximum(agg * dinv + b1_ref[...], 0.0)
    t2 = lax.dot_general(h, w2_ref[...], (((1,), (1,)), ((), ())),
                         preferred_element_type=jnp.float32)
    u2_ref[...] = t2 * dinv


def _final_body(degp_ref, parts_ref, u2_ref, b2_ref, o_ref):
    agg = parts_ref[0] + parts_ref[1] - u2_ref[...]
    o_ref[...] = agg * _dinv(degp_ref) + b2_ref[...]


def _full(shape):
    return pl.BlockSpec(shape, lambda i: tuple(0 for _ in shape))


def _rows(d):
    return pl.BlockSpec((_BLK, d), lambda i: (i, 0))


_degs = pl.BlockSpec((NC, NP), lambda i: (0, 0))


def _parts(d2):
    return pl.BlockSpec((NC, _BLK, d2), lambda i: (0, i, 0))


def kernel(x, edge_index, W1, b1, W2, b2):
    e4 = edge_index.reshape(2, TW, K)
    xp = jnp.pad(x, ((0, NP - N), (0, 0)))
    b1r = b1.reshape(1, D_HID)
    b2r = b2.reshape(1, D_OUT)

    degp = _deg_kernel(e4)

    u1 = pl.pallas_call(
        _layer1_body,
        grid=(_GRID,),
        in_specs=[_degs, _rows(D_IN), _full((D_HID, D_IN))],
        out_specs=_parts(_H2),
        out_shape=jax.ShapeDtypeStruct((NC, NP, _H2), jnp.float32),
        compiler_params=_tc_params,
    )(degp, xp, W1)

    parts1 = _prop128(u1, e4)

    u2 = pl.pallas_call(
        _layer2_body,
        grid=(_GRID,),
        in_specs=[_degs, _parts(_H2), _full((1, D_HID)),
                  _full((D_OUT, D_HID))],
        out_specs=_rows(D_OUT),
        out_shape=jax.ShapeDtypeStruct((NP, D_OUT), jnp.float32),
        compiler_params=_tc_params,
    )(degp, parts1, b1r, W2)

    parts2 = _prop64(u2, e4)

    outp = pl.pallas_call(
        _final_body,
        grid=(_GRID,),
        in_specs=[_degs,
                  pl.BlockSpec((NC, _BLK, D_OUT), lambda i: (0, i, 0)),
                  _rows(D_OUT), _full((1, D_OUT))],
        out_specs=_rows(D_OUT),
        out_shape=jax.ShapeDtypeStruct((NP, D_OUT), jnp.float32),
        compiler_params=_tc_params,
    )(degp, parts2, u2, b2r)

    return outp[:N]
